# baseline (speedup 1.0000x reference)
import functools

import jax
import jax.numpy as jnp
from jax import lax
from jax.experimental import pallas as pl
from jax.experimental.pallas import tpu as pltpu
from jax.experimental.pallas import tpu_sc as plsc

_B, _T, _H, _W, _C = 16, 16, 14, 14, 768
_NSLAB = _B * _H * _W
_ROWS = _NSLAB * _T
_NW = 32
_SPW = _NSLAB // _NW
_RPW = _ROWS // _NW
_CHUNK = 56
_NCHUNK = _RPW // _CHUNK


@functools.cache
def _build_sc_patch_shift():
    @functools.partial(
        pl.kernel,
        mesh=plsc.VectorSubcoreMesh(core_axis_name="c", subcore_axis_name="s"),
        out_type=jax.ShapeDtypeStruct((_ROWS, _C), jnp.float32),
        scratch_types=[
            pltpu.VMEM((_RPW,), jnp.int32),
            pltpu.VMEM((_CHUNK, _C), jnp.float32),
            pltpu.VMEM((_CHUNK, _C), jnp.float32),
            pltpu.SemaphoreType.DMA,
            pltpu.SemaphoreType.DMA,
        ],
    )
    def _sc_patch_shift(x_hbm, out_hbm, idx_v, buf0, buf1, gs0, gs1):
        wid = lax.axis_index("s") * 2 + lax.axis_index("c")
        base = wid * _RPW
        sbase = wid * _SPW
        lanes = lax.iota(jnp.int32, 16)

        def idx_body(i, carry):
            slab = sbase + i
            q = lax.div(slab, _W)
            w = slab - q * _W
            h = q - lax.div(q, _H) * _H
            p = h * _W + w
            h7 = lax.div(p, 7)
            w7 = p - h7 * 7
            code = (w7 % 3) * 3 + (h7 % 3)
            s = jnp.where(code == 0, -4,
                jnp.where(code == 1, 1,
                jnp.where(code == 2, 2,
                jnp.where(code == 3, -1,
                jnp.where(code == 5, 3,
                jnp.where(code == 6, -2,
                jnp.where(code == 7, -3,
                jnp.where(code == 8, 4,
                    jnp.where(p == 8, 0, -1)))))))))
            idx_v[pl.ds(i * 16, 16)] = slab * _T + ((lanes - s + _T) & (_T - 1))
            return carry

        lax.fori_loop(0, _SPW, idx_body, 0)

        def start_gather(c, buf, sem):
            pltpu.async_copy(x_hbm.at[idx_v.at[pl.ds(c * _CHUNK, _CHUNK)]], buf, sem)

        def wait_gather(c, buf, sem):
            pltpu.make_async_copy(
                x_hbm.at[idx_v.at[pl.ds(c * _CHUNK, _CHUNK)]], buf, sem).wait()

        def scatter(c, buf):
            pltpu.sync_copy(buf, out_hbm.at[pl.ds(base + c * _CHUNK, _CHUNK)])

        start_gather(0, buf0, gs0)
        start_gather(1, buf1, gs1)

        def body(i, carry):
            g = 2 * i
            wait_gather(g, buf0, gs0)
            scatter(g, buf0)
            start_gather(g + 2, buf0, gs0)
            wait_gather(g + 1, buf1, gs1)
            scatter(g + 1, buf1)
            start_gather(g + 3, buf1, gs1)
            return carry

        lax.fori_loop(0, (_NCHUNK - 2) // 2, body, 0)

        g = _NCHUNK - 2
        wait_gather(g, buf0, gs0)
        scatter(g, buf0)
        wait_gather(g + 1, buf1, gs1)
        scatter(g + 1, buf1)

    return _sc_patch_shift


def kernel(x):
    xl = x.transpose(0, 2, 3, 1, 4).reshape(_ROWS, _C)
    out = _build_sc_patch_shift()(xl)
    return out.reshape(_B, _H, _W, _T, _C).transpose(0, 3, 1, 2, 4)
